# baseline (speedup 1.0000x reference)
import jax
import jax.numpy as jnp
from jax import lax
from jax.experimental import pallas as pl
from jax.experimental.pallas import tpu as pltpu
from jax.experimental.pallas import tpu_sc as plsc

_N = 1000
_NP = 1024
_E = 100
_EP = 128
_D = 256
_FPT = 16

_DN = (((1,), (0,)), ((), ()))


def _prep(eint):
    src_c = eint[:, 0:1]
    dst_c = eint[:, 1:2]
    val_c = eint[:, 2:3]
    ew_c = (src_c != dst_c) & (val_c == 1)

    iota_en = lax.broadcasted_iota(jnp.int32, (_EP, _NP), 1)
    one = jnp.float32(1.0)
    zero = jnp.float32(0.0)
    oh_dstT = jnp.where((iota_en == dst_c) & ew_c, one, zero)

    deg = 1.0 + jnp.sum(oh_dstT, axis=0, keepdims=True)
    dis = lax.rsqrt(deg)
    dinv = 1.0 / deg

    dis_src = jnp.sum(jnp.where((iota_en == src_c) & ew_c, dis, zero),
                      axis=1, keepdims=True)
    dis_dst = jnp.sum(jnp.where((iota_en == dst_c) & ew_c, dis, zero),
                      axis=1, keepdims=True)
    norm_c = dis_src * dis_dst
    return dinv, norm_c


def _tc_first_body(eint_ref, xt_ref, w_ref, b_ref,
                   ht_ref, hpret_ref, dinv_ref, norm_ref):
    dinv, norm_c = _prep(eint_ref[...])
    ht = lax.dot_general(w_ref[...], xt_ref[...], _DN,
                         preferred_element_type=jnp.float32)
    ht_ref[...] = ht
    hpret_ref[...] = dinv * ht + b_ref[...]
    dinv_ref[...] = dinv
    norm_ref[...] = norm_c


def _tc_mid_body(yt_ref, w_ref, b_ref, dinv_ref, ht_ref, hpret_ref):
    xt = jnp.maximum(yt_ref[...], 0.0)
    ht = lax.dot_general(w_ref[...], xt, _DN,
                         preferred_element_type=jnp.float32)
    ht_ref[...] = ht
    hpret_ref[...] = dinv_ref[...] * ht + b_ref[...]


_PIB = lax.GatherScatterMode.PROMISE_IN_BOUNDS
_GDN = lax.GatherDimensionNumbers(
    offset_dims=(), collapsed_slice_dims=(0,), start_index_map=(0,))


def _splat(vec, j):
    idx = jnp.full((16, 1), j, jnp.int32)
    return lax.gather(vec, idx, _GDN, slice_sizes=(1,), mode=_PIB)


def _sc_agg_body(ht_hbm, hpret_hbm, src_hbm, dst_hbm, norm_hbm,
                 yt_hbm, src_v, dst_v, norm_v, hloc_v, acc_v):
    c = lax.axis_index("c")
    s = lax.axis_index("s")

    @pl.when(c == 0)
    def _():
        for r in range(_FPT):
            row = pl.ds(r * _NP, _NP)
            pltpu.sync_copy(ht_hbm.at[s * _FPT + r], hloc_v.at[row])
            pltpu.sync_copy(hpret_hbm.at[s * _FPT + r], acc_v.at[row])
        pltpu.sync_copy(src_hbm, src_v)
        pltpu.sync_copy(dst_hbm, dst_v)
        pltpu.sync_copy(norm_hbm, norm_v)

        lanebase = lax.iota(jnp.int32, 16) * _NP
        for ev in range(_EP // 16):
            sl = pl.ds(ev * 16, 16)
            srcv = src_v[sl]
            dstv = dst_v[sl]
            normv = norm_v[sl]
            for j in range(16):
                sj = _splat(srcv, j)
                dj = _splat(dstv, j)
                nj = _splat(normv, j)
                g = plsc.load_gather(hloc_v, [lanebase + sj])
                plsc.addupdate_scatter(acc_v, [lanebase + dj], g * nj)

        for r in range(_FPT):
            row = pl.ds(r * _NP, _NP)
            pltpu.sync_copy(acc_v.at[row], yt_hbm.at[s * _FPT + r])


_sc_agg = pl.kernel(
    _sc_agg_body,
    out_type=jax.ShapeDtypeStruct((_D, _NP), jnp.float32),
    mesh=plsc.VectorSubcoreMesh(core_axis_name="c", subcore_axis_name="s"),
    compiler_params=pltpu.CompilerParams(use_tc_tiling_on_sc=False,
                                         needs_layout_passes=False),
    scratch_types=[
        pltpu.VMEM((_EP,), jnp.int32),
        pltpu.VMEM((_EP,), jnp.int32),
        pltpu.VMEM((_EP,), jnp.float32),
        pltpu.VMEM((_FPT * _NP,), jnp.float32),
        pltpu.VMEM((_FPT * _NP,), jnp.float32),
    ],
)

_tc_first = pl.pallas_call(
    _tc_first_body,
    out_shape=(
        jax.ShapeDtypeStruct((_D, _NP), jnp.float32),
        jax.ShapeDtypeStruct((_D, _NP), jnp.float32),
        jax.ShapeDtypeStruct((1, _NP), jnp.float32),
        jax.ShapeDtypeStruct((_EP, 1), jnp.float32),
    ),
)

_tc_mid = pl.pallas_call(
    _tc_mid_body,
    out_shape=(
        jax.ShapeDtypeStruct((_D, _NP), jnp.float32),
        jax.ShapeDtypeStruct((_D, _NP), jnp.float32),
    ),
)


def kernel(L_x_, L_edge_index_,
           L_self_modules_convs_modules_0_modules_lin_parameters_weight_,
           L_self_modules_convs_modules_0_parameters_bias_,
           L_self_modules_convs_modules_1_modules_lin_parameters_weight_,
           L_self_modules_convs_modules_1_parameters_bias_,
           L_self_modules_convs_modules_2_modules_lin_parameters_weight_,
           L_self_modules_convs_modules_2_parameters_bias_,
           L_self_modules_convs_modules_3_modules_lin_parameters_weight_,
           L_self_modules_convs_modules_3_parameters_bias_,
           L_self_modules_convs_modules_4_modules_lin_parameters_weight_,
           L_self_modules_convs_modules_4_parameters_bias_):
    ws = [L_self_modules_convs_modules_0_modules_lin_parameters_weight_,
          L_self_modules_convs_modules_1_modules_lin_parameters_weight_,
          L_self_modules_convs_modules_2_modules_lin_parameters_weight_,
          L_self_modules_convs_modules_3_modules_lin_parameters_weight_,
          L_self_modules_convs_modules_4_modules_lin_parameters_weight_]
    bs = [L_self_modules_convs_modules_0_parameters_bias_,
          L_self_modules_convs_modules_1_parameters_bias_,
          L_self_modules_convs_modules_2_parameters_bias_,
          L_self_modules_convs_modules_3_parameters_bias_,
          L_self_modules_convs_modules_4_parameters_bias_]

    xt = jnp.pad(L_x_.T, ((0, 0), (0, _NP - _N)))
    e = L_edge_index_.astype(jnp.int32)
    e = jnp.pad(e, ((0, 0), (0, _EP - _E)))
    valid = (jnp.arange(_EP, dtype=jnp.int32) < _E).astype(jnp.int32)
    eint = jnp.zeros((_EP, 8), jnp.int32)
    eint = eint.at[:, 0].set(e[0]).at[:, 1].set(e[1]).at[:, 2].set(valid)
    src = e[0]
    dst = e[1]

    bcs = [b.reshape(_D, 1) for b in bs]

    ht, hpret, dinv, norm_c = _tc_first(eint, xt, ws[0], bcs[0])
    norm = norm_c.reshape(_EP)
    yt = _sc_agg(ht, hpret, src, dst, norm)
    for i in range(1, 5):
        ht, hpret = _tc_mid(yt, ws[i], bcs[i], dinv)
        yt = _sc_agg(ht, hpret, src, dst, norm)
    return yt[:, :_N].T


# speedup vs baseline: 1.8118x; 1.8118x over previous
import jax
import jax.numpy as jnp
from jax import lax
from jax.experimental import pallas as pl
from jax.experimental.pallas import tpu as pltpu
from jax.experimental.pallas import tpu_sc as plsc

_N = 1000
_NP = 1024
_E = 100
_EP = 128
_D = 256
_FPT = 16

_DN = (((1,), (0,)), ((), ()))


def _prep(eint):
    src_c = eint[:, 0:1]
    dst_c = eint[:, 1:2]
    val_c = eint[:, 2:3]
    ew_c = (src_c != dst_c) & (val_c == 1)

    iota_en = lax.broadcasted_iota(jnp.int32, (_EP, _NP), 1)
    one = jnp.float32(1.0)
    zero = jnp.float32(0.0)
    oh_dstT = jnp.where((iota_en == dst_c) & ew_c, one, zero)

    deg = 1.0 + jnp.sum(oh_dstT, axis=0, keepdims=True)
    dis = lax.rsqrt(deg)
    dinv = 1.0 / deg

    dis_src = jnp.sum(jnp.where((iota_en == src_c) & ew_c, dis, zero),
                      axis=1, keepdims=True)
    dis_dst = jnp.sum(jnp.where((iota_en == dst_c) & ew_c, dis, zero),
                      axis=1, keepdims=True)
    norm_c = dis_src * dis_dst
    return dinv, norm_c


def _tc_first_body(eint_ref, xt_ref, w_ref, b_ref,
                   ht_ref, hpret_ref, dinv_ref, norm_ref):
    dinv, norm_c = _prep(eint_ref[...])
    ht = lax.dot_general(w_ref[...], xt_ref[...], _DN,
                         preferred_element_type=jnp.float32)
    ht_ref[...] = ht
    hpret_ref[...] = dinv * ht + b_ref[...]
    dinv_ref[...] = dinv
    norm_ref[...] = norm_c


def _tc_mid_body(yt_ref, w_ref, b_ref, dinv_ref, ht_ref, hpret_ref):
    xt = jnp.maximum(yt_ref[...], 0.0)
    ht = lax.dot_general(w_ref[...], xt, _DN,
                         preferred_element_type=jnp.float32)
    ht_ref[...] = ht
    hpret_ref[...] = dinv_ref[...] * ht + b_ref[...]


_PIB = lax.GatherScatterMode.PROMISE_IN_BOUNDS
_GDN = lax.GatherDimensionNumbers(
    offset_dims=(), collapsed_slice_dims=(0,), start_index_map=(0,))


def _splat(vec, j):
    idx = jnp.full((16, 1), j, jnp.int32)
    return lax.gather(vec, idx, _GDN, slice_sizes=(1,), mode=_PIB)


def _sc_agg_body(ht_hbm, hpret_hbm, src_hbm, dst_hbm, norm_hbm,
                 yt_hbm, src_v, dst_v, norm_v, hloc_v, acc_v, sem):
    c = lax.axis_index("c")
    s = lax.axis_index("s")

    @pl.when(c == 0)
    def _():
        cps = []
        for r in range(_FPT):
            row = pl.ds(r * _NP, _NP)
            cps.append(pltpu.async_copy(
                ht_hbm.at[s * _FPT + r], hloc_v.at[row], sem))
            cps.append(pltpu.async_copy(
                hpret_hbm.at[s * _FPT + r], acc_v.at[row], sem))
        cps.append(pltpu.async_copy(src_hbm, src_v, sem))
        cps.append(pltpu.async_copy(dst_hbm, dst_v, sem))
        cps.append(pltpu.async_copy(norm_hbm, norm_v, sem))
        for cp in cps:
            cp.wait()

        lanebase = lax.iota(jnp.int32, 16) * _NP
        for ev in range(_EP // 16):
            sl = pl.ds(ev * 16, 16)
            srcv = src_v[sl]
            dstv = dst_v[sl]
            normv = norm_v[sl]
            for j in range(16):
                sj = _splat(srcv, j)
                dj = _splat(dstv, j)
                nj = _splat(normv, j)
                g = plsc.load_gather(hloc_v, [lanebase + sj])
                plsc.addupdate_scatter(acc_v, [lanebase + dj], g * nj)

        ops = []
        for r in range(_FPT):
            row = pl.ds(r * _NP, _NP)
            ops.append(pltpu.async_copy(
                acc_v.at[row], yt_hbm.at[s * _FPT + r], sem))
        for op in ops:
            op.wait()


_sc_agg = pl.kernel(
    _sc_agg_body,
    out_type=jax.ShapeDtypeStruct((_D, _NP), jnp.float32),
    mesh=plsc.VectorSubcoreMesh(core_axis_name="c", subcore_axis_name="s"),
    compiler_params=pltpu.CompilerParams(use_tc_tiling_on_sc=False,
                                         needs_layout_passes=False),
    scratch_types=[
        pltpu.VMEM((_EP,), jnp.int32),
        pltpu.VMEM((_EP,), jnp.int32),
        pltpu.VMEM((_EP,), jnp.float32),
        pltpu.VMEM((_FPT * _NP,), jnp.float32),
        pltpu.VMEM((_FPT * _NP,), jnp.float32),
        pltpu.SemaphoreType.DMA,
    ],
)

_tc_first = pl.pallas_call(
    _tc_first_body,
    out_shape=(
        jax.ShapeDtypeStruct((_D, _NP), jnp.float32),
        jax.ShapeDtypeStruct((_D, _NP), jnp.float32),
        jax.ShapeDtypeStruct((1, _NP), jnp.float32),
        jax.ShapeDtypeStruct((_EP, 1), jnp.float32),
    ),
)

_tc_mid = pl.pallas_call(
    _tc_mid_body,
    out_shape=(
        jax.ShapeDtypeStruct((_D, _NP), jnp.float32),
        jax.ShapeDtypeStruct((_D, _NP), jnp.float32),
    ),
)


def kernel(L_x_, L_edge_index_,
           L_self_modules_convs_modules_0_modules_lin_parameters_weight_,
           L_self_modules_convs_modules_0_parameters_bias_,
           L_self_modules_convs_modules_1_modules_lin_parameters_weight_,
           L_self_modules_convs_modules_1_parameters_bias_,
           L_self_modules_convs_modules_2_modules_lin_parameters_weight_,
           L_self_modules_convs_modules_2_parameters_bias_,
           L_self_modules_convs_modules_3_modules_lin_parameters_weight_,
           L_self_modules_convs_modules_3_parameters_bias_,
           L_self_modules_convs_modules_4_modules_lin_parameters_weight_,
           L_self_modules_convs_modules_4_parameters_bias_):
    ws = [L_self_modules_convs_modules_0_modules_lin_parameters_weight_,
          L_self_modules_convs_modules_1_modules_lin_parameters_weight_,
          L_self_modules_convs_modules_2_modules_lin_parameters_weight_,
          L_self_modules_convs_modules_3_modules_lin_parameters_weight_,
          L_self_modules_convs_modules_4_modules_lin_parameters_weight_]
    bs = [L_self_modules_convs_modules_0_parameters_bias_,
          L_self_modules_convs_modules_1_parameters_bias_,
          L_self_modules_convs_modules_2_parameters_bias_,
          L_self_modules_convs_modules_3_parameters_bias_,
          L_self_modules_convs_modules_4_parameters_bias_]

    xt = jnp.pad(L_x_.T, ((0, 0), (0, _NP - _N)))
    e = L_edge_index_.astype(jnp.int32)
    e = jnp.pad(e, ((0, 0), (0, _EP - _E)))
    valid = (jnp.arange(_EP, dtype=jnp.int32) < _E).astype(jnp.int32)
    eint = jnp.zeros((_EP, 8), jnp.int32)
    eint = eint.at[:, 0].set(e[0]).at[:, 1].set(e[1]).at[:, 2].set(valid)
    src = e[0]
    dst = e[1]

    bcs = [b.reshape(_D, 1) for b in bs]

    ht, hpret, dinv, norm_c = _tc_first(eint, xt, ws[0], bcs[0])
    norm = norm_c.reshape(_EP)
    yt = _sc_agg(ht, hpret, src, dst, norm)
    for i in range(1, 5):
        ht, hpret = _tc_mid(yt, ws[i], bcs[i], dinv)
        yt = _sc_agg(ht, hpret, src, dst, norm)
    return yt[:, :_N].T


# speedup vs baseline: 2.3291x; 1.2856x over previous
import jax
import jax.numpy as jnp
from jax import lax
from jax.experimental import pallas as pl
from jax.experimental.pallas import tpu as pltpu
from jax.experimental.pallas import tpu_sc as plsc

_N = 1000
_NP = 1024
_E = 100
_EP = 128
_D = 256
_FPT = 16
_G = _NP // 128

_DN = (((1,), (0,)), ((), ()))
_DNT = (((1,), (1,)), ((), ()))


def _tc_first_body(eint_ref, x_ref, w_ref, b_ref,
                   h3_ref, hp3_ref, dinv8_ref, norm_ref):
    eint = eint_ref[...]
    src_c = eint[:, 0:1]
    dst_c = eint[:, 1:2]
    ew_c = src_c != dst_c

    iota_en = lax.broadcasted_iota(jnp.int32, (_E, _NP), 1)
    zero = jnp.float32(0.0)
    oh_dst = jnp.where((iota_en == dst_c) & ew_c, jnp.float32(1.0), zero)

    deg = 1.0 + jnp.sum(oh_dst, axis=0, keepdims=True)
    dis = lax.rsqrt(deg)
    dinv = 1.0 / deg

    dis_src = jnp.sum(jnp.where((iota_en == src_c) & ew_c, dis, zero),
                      axis=1, keepdims=True)
    dis_dst = jnp.sum(jnp.where((iota_en == dst_c) & ew_c, dis, zero),
                      axis=1, keepdims=True)
    norm_ref[...] = jnp.pad(dis_src * dis_dst, ((0, _EP - _E), (0, 0)))

    ht = lax.dot_general(w_ref[...], x_ref[...], _DNT,
                         preferred_element_type=jnp.float32)
    ht = jnp.pad(ht, ((0, 0), (0, _NP - _N)))
    hpre = dinv * ht + b_ref[...]
    for g in range(_G):
        sl = slice(g * 128, (g + 1) * 128)
        h3_ref[:, g, :] = ht[:, sl]
        hp3_ref[:, g, :] = hpre[:, sl]
        dinv8_ref[pl.ds(g, 1), :] = dinv[:, sl]


def _tc_mid_body(x3_ref, w_ref, b_ref, dinv8_ref, h3_ref, hp3_ref):
    xt = jnp.concatenate(
        [jnp.maximum(x3_ref[:, g, :], 0.0) for g in range(_G)], axis=1)
    ht = lax.dot_general(w_ref[...], xt, _DN,
                         preferred_element_type=jnp.float32)
    dinv = jnp.concatenate(
        [dinv8_ref[pl.ds(g, 1), :] for g in range(_G)], axis=1)
    hpre = dinv * ht + b_ref[...]
    for g in range(_G):
        sl = slice(g * 128, (g + 1) * 128)
        h3_ref[:, g, :] = ht[:, sl]
        hp3_ref[:, g, :] = hpre[:, sl]


_PIB = lax.GatherScatterMode.PROMISE_IN_BOUNDS
_GDN = lax.GatherDimensionNumbers(
    offset_dims=(), collapsed_slice_dims=(0,), start_index_map=(0,))


def _splat(vec, j):
    idx = jnp.full((16, 1), j, jnp.int32)
    return lax.gather(vec, idx, _GDN, slice_sizes=(1,), mode=_PIB)


def _sc_agg_body(ht_hbm, hpret_hbm, src_hbm, dst_hbm, norm_hbm,
                 yt_hbm, src_v, dst_v, norm_v, hloc_v, acc_v, sem):
    c = lax.axis_index("c")
    s = lax.axis_index("s")

    @pl.when(c == 0)
    def _():
        base = s * (_FPT * _NP)
        chunk = pl.ds(base, _FPT * _NP)
        cps = [
            pltpu.async_copy(ht_hbm.at[chunk], hloc_v, sem),
            pltpu.async_copy(hpret_hbm.at[chunk], acc_v, sem),
            pltpu.async_copy(src_hbm, src_v, sem),
            pltpu.async_copy(dst_hbm, dst_v, sem),
            pltpu.async_copy(norm_hbm, norm_v, sem),
        ]
        for cp in cps:
            cp.wait()

        lanebase = lax.iota(jnp.int32, 16) * _NP
        for ev in range(_EP // 16):
            sl = pl.ds(ev * 16, 16)
            srcv = src_v[sl]
            dstv = dst_v[sl]
            normv = norm_v[sl]
            for j in range(16):
                sj = _splat(srcv, j)
                dj = _splat(dstv, j)
                nj = _splat(normv, j)
                g = plsc.load_gather(hloc_v, [lanebase + sj])
                plsc.addupdate_scatter(acc_v, [lanebase + dj], g * nj)

        pltpu.sync_copy(acc_v, yt_hbm.at[chunk])


_sc_agg = pl.kernel(
    _sc_agg_body,
    out_type=jax.ShapeDtypeStruct((_D * _NP,), jnp.float32),
    mesh=plsc.VectorSubcoreMesh(core_axis_name="c", subcore_axis_name="s"),
    compiler_params=pltpu.CompilerParams(use_tc_tiling_on_sc=False,
                                         needs_layout_passes=False),
    scratch_types=[
        pltpu.VMEM((_EP,), jnp.int32),
        pltpu.VMEM((_EP,), jnp.int32),
        pltpu.VMEM((_EP,), jnp.float32),
        pltpu.VMEM((_FPT * _NP,), jnp.float32),
        pltpu.VMEM((_FPT * _NP,), jnp.float32),
        pltpu.SemaphoreType.DMA,
    ],
)

_tc_first = pl.pallas_call(
    _tc_first_body,
    out_shape=(
        jax.ShapeDtypeStruct((_D, _G, 128), jnp.float32),
        jax.ShapeDtypeStruct((_D, _G, 128), jnp.float32),
        jax.ShapeDtypeStruct((_G, 128), jnp.float32),
        jax.ShapeDtypeStruct((_EP, 1), jnp.float32),
    ),
)

_tc_mid = pl.pallas_call(
    _tc_mid_body,
    out_shape=(
        jax.ShapeDtypeStruct((_D, _G, 128), jnp.float32),
        jax.ShapeDtypeStruct((_D, _G, 128), jnp.float32),
    ),
)


def kernel(L_x_, L_edge_index_,
           L_self_modules_convs_modules_0_modules_lin_parameters_weight_,
           L_self_modules_convs_modules_0_parameters_bias_,
           L_self_modules_convs_modules_1_modules_lin_parameters_weight_,
           L_self_modules_convs_modules_1_parameters_bias_,
           L_self_modules_convs_modules_2_modules_lin_parameters_weight_,
           L_self_modules_convs_modules_2_parameters_bias_,
           L_self_modules_convs_modules_3_modules_lin_parameters_weight_,
           L_self_modules_convs_modules_3_parameters_bias_,
           L_self_modules_convs_modules_4_modules_lin_parameters_weight_,
           L_self_modules_convs_modules_4_parameters_bias_):
    ws = [L_self_modules_convs_modules_0_modules_lin_parameters_weight_,
          L_self_modules_convs_modules_1_modules_lin_parameters_weight_,
          L_self_modules_convs_modules_2_modules_lin_parameters_weight_,
          L_self_modules_convs_modules_3_modules_lin_parameters_weight_,
          L_self_modules_convs_modules_4_modules_lin_parameters_weight_]
    bs = [L_self_modules_convs_modules_0_parameters_bias_,
          L_self_modules_convs_modules_1_parameters_bias_,
          L_self_modules_convs_modules_2_parameters_bias_,
          L_self_modules_convs_modules_3_parameters_bias_,
          L_self_modules_convs_modules_4_parameters_bias_]

    e = L_edge_index_.astype(jnp.int32)
    eint = jnp.pad(e.T, ((0, 0), (0, 6)))
    src = jnp.pad(e[0], (0, _EP - _E))
    dst = jnp.pad(e[1], (0, _EP - _E))
    bcs = [b.reshape(_D, 1) for b in bs]

    h3, hp3, dinv8, norm_c = _tc_first(eint, L_x_, ws[0], bcs[0])
    norm = norm_c.reshape(_EP)
    yt = _sc_agg(h3.reshape(-1), hp3.reshape(-1), src, dst, norm)
    for i in range(1, 5):
        h3, hp3 = _tc_mid(yt.reshape(_D, _G, 128), ws[i], bcs[i], dinv8)
        yt = _sc_agg(h3.reshape(-1), hp3.reshape(-1), src, dst, norm)
    return yt.reshape(_D, _NP)[:, :_N].T


# speedup vs baseline: 2.5206x; 1.0822x over previous
import jax
import jax.numpy as jnp
from jax import lax
from jax.experimental import pallas as pl
from jax.experimental.pallas import tpu as pltpu
from jax.experimental.pallas import tpu_sc as plsc

_N = 1000
_NP = 1024
_E = 100
_EP = 128
_D = 256
_FPT = 16
_G = _NP // 128

_DN = (((1,), (0,)), ((), ()))
_DNT = (((1,), (1,)), ((), ()))


def _tc_first_body(eint_ref, x_ref, w_ref, b_ref,
                   h3_ref, hp3_ref, dinv8_ref, norm_ref):
    eint = eint_ref[...]
    src_c = eint[:, 0:1]
    dst_c = eint[:, 1:2]
    ew_c = src_c != dst_c

    iota_en = lax.broadcasted_iota(jnp.int32, (_E, _NP), 1)
    zero = jnp.float32(0.0)
    oh_dst = jnp.where((iota_en == dst_c) & ew_c, jnp.float32(1.0), zero)

    deg = 1.0 + jnp.sum(oh_dst, axis=0, keepdims=True)
    dis = lax.rsqrt(deg)
    dinv = 1.0 / deg

    dis_src = jnp.sum(jnp.where((iota_en == src_c) & ew_c, dis, zero),
                      axis=1, keepdims=True)
    dis_dst = jnp.sum(jnp.where((iota_en == dst_c) & ew_c, dis, zero),
                      axis=1, keepdims=True)
    norm_ref[...] = jnp.pad(dis_src * dis_dst, ((0, _EP - _E), (0, 0)))

    ht = lax.dot_general(w_ref[...], x_ref[...], _DNT,
                         preferred_element_type=jnp.float32)
    ht = jnp.pad(ht, ((0, 0), (0, _NP - _N)))
    hpre = dinv * ht + b_ref[...]
    for g in range(_G):
        sl = slice(g * 128, (g + 1) * 128)
        h3_ref[:, g, :] = ht[:, sl]
        hp3_ref[:, g, :] = hpre[:, sl]
        dinv8_ref[pl.ds(g, 1), :] = dinv[:, sl]


def _tc_mid_body(x3_ref, w_ref, b_ref, dinv8_ref, h3_ref, hp3_ref):
    xt = jnp.concatenate(
        [jnp.maximum(x3_ref[:, g, :], 0.0) for g in range(_G)], axis=1)
    ht = lax.dot_general(w_ref[...], xt, _DN,
                         preferred_element_type=jnp.float32)
    dinv = jnp.concatenate(
        [dinv8_ref[pl.ds(g, 1), :] for g in range(_G)], axis=1)
    hpre = dinv * ht + b_ref[...]
    for g in range(_G):
        sl = slice(g * 128, (g + 1) * 128)
        h3_ref[:, g, :] = ht[:, sl]
        hp3_ref[:, g, :] = hpre[:, sl]


_PIB = lax.GatherScatterMode.PROMISE_IN_BOUNDS
_GDN = lax.GatherDimensionNumbers(
    offset_dims=(), collapsed_slice_dims=(0,), start_index_map=(0,))


def _splat(vec, j):
    idx = jnp.full((16, 1), j, jnp.int32)
    return lax.gather(vec, idx, _GDN, slice_sizes=(1,), mode=_PIB)


_FPW = 8


def _sc_agg_body(ht_hbm, hpret_hbm, src_hbm, dst_hbm, norm_hbm,
                 yt_hbm, src_v, dst_v, norm_v, hloc_v, acc_v, sem):
    c = lax.axis_index("c")
    s = lax.axis_index("s")
    w = s * 2 + c

    rows = pl.ds(w * _FPW, _FPW)
    cps = [
        pltpu.async_copy(ht_hbm.at[rows], hloc_v, sem),
        pltpu.async_copy(hpret_hbm.at[rows], acc_v, sem),
        pltpu.async_copy(src_hbm, src_v, sem),
        pltpu.async_copy(dst_hbm, dst_v, sem),
        pltpu.async_copy(norm_hbm, norm_v, sem),
    ]
    for cp in cps:
        cp.wait()

    lane = lax.iota(jnp.int32, 16)
    lmask = lane < _FPW
    lrow = jnp.minimum(lane, _FPW - 1)
    for ev in range(_EP // 16):
        sl = pl.ds(ev * 16, 16)
        srcv = src_v[sl]
        dstv = dst_v[sl]
        normv = norm_v[sl]
        for j in range(16):
            sj = _splat(srcv, j)
            dj = _splat(dstv, j)
            nj = _splat(normv, j)
            g = plsc.load_gather(
                hloc_v, [lrow, sj >> 7, sj & 127], mask=lmask)
            plsc.addupdate_scatter(
                acc_v, [lrow, dj >> 7, dj & 127], g * nj, mask=lmask)

    pltpu.sync_copy(acc_v, yt_hbm.at[rows])


_sc_agg = pl.kernel(
    _sc_agg_body,
    out_type=jax.ShapeDtypeStruct((_D, _G, 128), jnp.float32),
    mesh=plsc.VectorSubcoreMesh(core_axis_name="c", subcore_axis_name="s"),
    compiler_params=pltpu.CompilerParams(use_tc_tiling_on_sc=False,
                                         needs_layout_passes=False),
    scratch_types=[
        pltpu.VMEM((_EP,), jnp.int32),
        pltpu.VMEM((_EP,), jnp.int32),
        pltpu.VMEM((_EP,), jnp.float32),
        pltpu.VMEM((_FPW, _G, 128), jnp.float32),
        pltpu.VMEM((_FPW, _G, 128), jnp.float32),
        pltpu.SemaphoreType.DMA,
    ],
)

_tc_first = pl.pallas_call(
    _tc_first_body,
    out_shape=(
        jax.ShapeDtypeStruct((_D, _G, 128), jnp.float32),
        jax.ShapeDtypeStruct((_D, _G, 128), jnp.float32),
        jax.ShapeDtypeStruct((_G, 128), jnp.float32),
        jax.ShapeDtypeStruct((_EP, 1), jnp.float32),
    ),
)

_tc_mid = pl.pallas_call(
    _tc_mid_body,
    out_shape=(
        jax.ShapeDtypeStruct((_D, _G, 128), jnp.float32),
        jax.ShapeDtypeStruct((_D, _G, 128), jnp.float32),
    ),
)


def kernel(L_x_, L_edge_index_,
           L_self_modules_convs_modules_0_modules_lin_parameters_weight_,
           L_self_modules_convs_modules_0_parameters_bias_,
           L_self_modules_convs_modules_1_modules_lin_parameters_weight_,
           L_self_modules_convs_modules_1_parameters_bias_,
           L_self_modules_convs_modules_2_modules_lin_parameters_weight_,
           L_self_modules_convs_modules_2_parameters_bias_,
           L_self_modules_convs_modules_3_modules_lin_parameters_weight_,
           L_self_modules_convs_modules_3_parameters_bias_,
           L_self_modules_convs_modules_4_modules_lin_parameters_weight_,
           L_self_modules_convs_modules_4_parameters_bias_):
    ws = [L_self_modules_convs_modules_0_modules_lin_parameters_weight_,
          L_self_modules_convs_modules_1_modules_lin_parameters_weight_,
          L_self_modules_convs_modules_2_modules_lin_parameters_weight_,
          L_self_modules_convs_modules_3_modules_lin_parameters_weight_,
          L_self_modules_convs_modules_4_modules_lin_parameters_weight_]
    bs = [L_self_modules_convs_modules_0_parameters_bias_,
          L_self_modules_convs_modules_1_parameters_bias_,
          L_self_modules_convs_modules_2_parameters_bias_,
          L_self_modules_convs_modules_3_parameters_bias_,
          L_self_modules_convs_modules_4_parameters_bias_]

    e = L_edge_index_.astype(jnp.int32)
    eint = jnp.pad(e.T, ((0, 0), (0, 6)))
    src = jnp.pad(e[0], (0, _EP - _E))
    dst = jnp.pad(e[1], (0, _EP - _E))
    bcs = [b.reshape(_D, 1) for b in bs]

    h3, hp3, dinv8, norm_c = _tc_first(eint, L_x_, ws[0], bcs[0])
    norm = norm_c.reshape(_EP)
    yt = _sc_agg(h3, hp3, src, dst, norm)
    for i in range(1, 5):
        h3, hp3 = _tc_mid(yt, ws[i], bcs[i], dinv8)
        yt = _sc_agg(h3, hp3, src, dst, norm)
    return yt.reshape(_D, _NP)[:, :_N].T


# speedup vs baseline: 2.7374x; 1.0860x over previous
import jax
import jax.numpy as jnp
from jax import lax
from jax.experimental import pallas as pl
from jax.experimental.pallas import tpu as pltpu
from jax.experimental.pallas import tpu_sc as plsc

_N = 1000
_NP = 1024
_E = 100
_EP = 128
_D = 256
_FPT = 16
_G = _NP // 128

_DN = (((1,), (0,)), ((), ()))
_DNT = (((1,), (1,)), ((), ()))


def _tc_first_body(eint_ref, x_ref, w_ref, b_ref,
                   h3_ref, hp3_ref, dinv8_ref, norm_ref):
    eint = eint_ref[...]
    src_c = eint[:, 0:1]
    dst_c = eint[:, 1:2]
    ew_c = src_c != dst_c

    iota_en = lax.broadcasted_iota(jnp.int32, (_E, _NP), 1)
    zero = jnp.float32(0.0)
    oh_dst = jnp.where((iota_en == dst_c) & ew_c, jnp.float32(1.0), zero)

    deg = 1.0 + jnp.sum(oh_dst, axis=0, keepdims=True)
    dis = lax.rsqrt(deg)
    dinv = 1.0 / deg

    dis_src = jnp.sum(jnp.where((iota_en == src_c) & ew_c, dis, zero),
                      axis=1, keepdims=True)
    dis_dst = jnp.sum(jnp.where((iota_en == dst_c) & ew_c, dis, zero),
                      axis=1, keepdims=True)
    norm_ref[...] = jnp.pad(dis_src * dis_dst, ((0, _EP - _E), (0, 0)))

    ht = lax.dot_general(w_ref[...], x_ref[...], _DNT,
                         preferred_element_type=jnp.float32)
    ht = jnp.pad(ht, ((0, 0), (0, _NP - _N)))
    hpre = dinv * ht + b_ref[...]
    for g in range(_G):
        sl = slice(g * 128, (g + 1) * 128)
        h3_ref[:, g, :, :] = ht[:, sl].reshape(_D // 8, 8, 128)
        hp3_ref[:, g, :, :] = hpre[:, sl].reshape(_D // 8, 8, 128)
        dinv8_ref[pl.ds(g, 1), :] = dinv[:, sl]


def _tc_mid_body(x3_ref, w_ref, b_ref, dinv8_ref, h3_ref, hp3_ref):
    xt = jnp.concatenate(
        [jnp.maximum(x3_ref[:, pl.ds(g, 1), :, :][...], 0.0)
         .reshape(_D, 128) for g in range(_G)], axis=1)
    ht = lax.dot_general(w_ref[...], xt, _DN,
                         preferred_element_type=jnp.float32)
    dinv = jnp.concatenate(
        [dinv8_ref[pl.ds(g, 1), :] for g in range(_G)], axis=1)
    hpre = dinv * ht + b_ref[...]
    for g in range(_G):
        sl = slice(g * 128, (g + 1) * 128)
        h3_ref[:, g, :, :] = ht[:, sl].reshape(_D // 8, 8, 128)
        hp3_ref[:, g, :, :] = hpre[:, sl].reshape(_D // 8, 8, 128)


_PIB = lax.GatherScatterMode.PROMISE_IN_BOUNDS
_GDN = lax.GatherDimensionNumbers(
    offset_dims=(), collapsed_slice_dims=(0,), start_index_map=(0,))


def _splat(vec, j):
    idx = jnp.full((16, 1), j, jnp.int32)
    return lax.gather(vec, idx, _GDN, slice_sizes=(1,), mode=_PIB)


_FPW = 8


def _sc_agg_body(ht_hbm, hpret_hbm, src_hbm, dst_hbm, norm_hbm,
                 yt_hbm, src_v, dst_v, norm_v, hloc_v, acc_v, sem):
    c = lax.axis_index("c")
    s = lax.axis_index("s")
    w = s * 2 + c
    cps = [
        pltpu.async_copy(ht_hbm.at[w], hloc_v, sem),
        pltpu.async_copy(hpret_hbm.at[w], acc_v, sem),
        pltpu.async_copy(src_hbm, src_v, sem),
        pltpu.async_copy(dst_hbm, dst_v, sem),
        pltpu.async_copy(norm_hbm, norm_v, sem),
    ]
    for cp in cps:
        cp.wait()

    lane = lax.iota(jnp.int32, 16)
    lmask = lane < _FPW
    lrow = jnp.minimum(lane, _FPW - 1)
    for ev in range(_EP // 16):
        sl = pl.ds(ev * 16, 16)
        srcv = src_v[sl]
        dstv = dst_v[sl]
        normv = norm_v[sl]
        for j in range(16):
            sj = _splat(srcv, j)
            dj = _splat(dstv, j)
            nj = _splat(normv, j)
            g = plsc.load_gather(
                hloc_v, [sj >> 7, lrow, sj & 127], mask=lmask)
            plsc.addupdate_scatter(
                acc_v, [dj >> 7, lrow, dj & 127], g * nj, mask=lmask)

    pltpu.sync_copy(acc_v, yt_hbm.at[w])


_sc_agg = pl.kernel(
    _sc_agg_body,
    out_type=jax.ShapeDtypeStruct((_D // 8, _G, 8, 128), jnp.float32),
    mesh=plsc.VectorSubcoreMesh(core_axis_name="c", subcore_axis_name="s"),
    compiler_params=pltpu.CompilerParams(use_tc_tiling_on_sc=False,
                                         needs_layout_passes=False),
    scratch_types=[
        pltpu.VMEM((_EP,), jnp.int32),
        pltpu.VMEM((_EP,), jnp.int32),
        pltpu.VMEM((_EP,), jnp.float32),
        pltpu.VMEM((_G, _FPW, 128), jnp.float32),
        pltpu.VMEM((_G, _FPW, 128), jnp.float32),
        pltpu.SemaphoreType.DMA,
    ],
)

_tc_first = pl.pallas_call(
    _tc_first_body,
    out_shape=(
        jax.ShapeDtypeStruct((_D // 8, _G, 8, 128), jnp.float32),
        jax.ShapeDtypeStruct((_D // 8, _G, 8, 128), jnp.float32),
        jax.ShapeDtypeStruct((_G, 128), jnp.float32),
        jax.ShapeDtypeStruct((_EP, 1), jnp.float32),
    ),
)

_tc_mid = pl.pallas_call(
    _tc_mid_body,
    out_shape=(
        jax.ShapeDtypeStruct((_D // 8, _G, 8, 128), jnp.float32),
        jax.ShapeDtypeStruct((_D // 8, _G, 8, 128), jnp.float32),
    ),
)


def kernel(L_x_, L_edge_index_,
           L_self_modules_convs_modules_0_modules_lin_parameters_weight_,
           L_self_modules_convs_modules_0_parameters_bias_,
           L_self_modules_convs_modules_1_modules_lin_parameters_weight_,
           L_self_modules_convs_modules_1_parameters_bias_,
           L_self_modules_convs_modules_2_modules_lin_parameters_weight_,
           L_self_modules_convs_modules_2_parameters_bias_,
           L_self_modules_convs_modules_3_modules_lin_parameters_weight_,
           L_self_modules_convs_modules_3_parameters_bias_,
           L_self_modules_convs_modules_4_modules_lin_parameters_weight_,
           L_self_modules_convs_modules_4_parameters_bias_):
    ws = [L_self_modules_convs_modules_0_modules_lin_parameters_weight_,
          L_self_modules_convs_modules_1_modules_lin_parameters_weight_,
          L_self_modules_convs_modules_2_modules_lin_parameters_weight_,
          L_self_modules_convs_modules_3_modules_lin_parameters_weight_,
          L_self_modules_convs_modules_4_modules_lin_parameters_weight_]
    bs = [L_self_modules_convs_modules_0_parameters_bias_,
          L_self_modules_convs_modules_1_parameters_bias_,
          L_self_modules_convs_modules_2_parameters_bias_,
          L_self_modules_convs_modules_3_parameters_bias_,
          L_self_modules_convs_modules_4_parameters_bias_]

    e = L_edge_index_.astype(jnp.int32)
    eint = jnp.pad(e.T, ((0, 0), (0, 6)))
    src = jnp.pad(e[0], (0, _EP - _E))
    dst = jnp.pad(e[1], (0, _EP - _E))
    bcs = [b.reshape(_D, 1) for b in bs]

    h3, hp3, dinv8, norm_c = _tc_first(eint, L_x_, ws[0], bcs[0])
    norm = norm_c.reshape(_EP)
    yt = _sc_agg(h3, hp3, src, dst, norm)
    for i in range(1, 5):
        h3, hp3 = _tc_mid(yt, ws[i], bcs[i], dinv8)
        yt = _sc_agg(h3, hp3, src, dst, norm)
    ytt = yt.transpose(0, 2, 1, 3).reshape(_D, _NP)
    return ytt[:, :_N].T


# speedup vs baseline: 2.9196x; 1.0665x over previous
import jax
import jax.numpy as jnp
from jax import lax
from jax.experimental import pallas as pl
from jax.experimental.pallas import tpu as pltpu
from jax.experimental.pallas import tpu_sc as plsc

_N = 1000
_NP = 1024
_E = 100
_EP = 128
_D = 256
_FPT = 16
_G = _NP // 128

_DN = (((1,), (0,)), ((), ()))
_DNT = (((1,), (1,)), ((), ()))


def _tc_first_body(eint_ref, x_ref, w_ref, b_ref,
                   h3_ref, hp3_ref, dinv8_ref, norm_ref,
                   src_ref, dst_ref):
    eint = eint_ref[...]
    src_c = eint[:, 0:1]
    dst_c = eint[:, 1:2]
    ew_c = src_c != dst_c
    src_ref[...] = jnp.pad(src_c, ((0, _EP - _E), (0, 0))).reshape(_EP)
    dst_ref[...] = jnp.pad(dst_c, ((0, _EP - _E), (0, 0))).reshape(_EP)

    iota_en = lax.broadcasted_iota(jnp.int32, (_E, _NP), 1)
    zero = jnp.float32(0.0)
    oh_dst = jnp.where((iota_en == dst_c) & ew_c, jnp.float32(1.0), zero)

    deg = 1.0 + jnp.sum(oh_dst, axis=0, keepdims=True)
    dis = lax.rsqrt(deg)
    dinv = 1.0 / deg

    dis_src = jnp.sum(jnp.where((iota_en == src_c) & ew_c, dis, zero),
                      axis=1, keepdims=True)
    dis_dst = jnp.sum(jnp.where((iota_en == dst_c) & ew_c, dis, zero),
                      axis=1, keepdims=True)
    norm_ref[...] = jnp.pad(dis_src * dis_dst,
                            ((0, _EP - _E), (0, 0))).reshape(_EP)

    ht = lax.dot_general(w_ref[...], x_ref[...], _DNT,
                         preferred_element_type=jnp.float32)
    ht = jnp.pad(ht, ((0, 0), (0, _NP - _N)))
    hpre = dinv * ht + b_ref[...].reshape(_D, 1)
    for g in range(_G):
        sl = slice(g * 128, (g + 1) * 128)
        h3_ref[:, g, :, :] = ht[:, sl].reshape(_D // 8, 8, 128)
        hp3_ref[:, g, :, :] = hpre[:, sl].reshape(_D // 8, 8, 128)
        dinv8_ref[pl.ds(g, 1), :] = dinv[:, sl]


def _tc_mid_body(x3_ref, w_ref, b_ref, dinv8_ref, h3_ref, hp3_ref):
    xt = jnp.concatenate(
        [jnp.maximum(x3_ref[:, pl.ds(g, 1), :, :][...], 0.0)
         .reshape(_D, 128) for g in range(_G)], axis=1)
    ht = lax.dot_general(w_ref[...], xt, _DN,
                         preferred_element_type=jnp.float32)
    dinv = jnp.concatenate(
        [dinv8_ref[pl.ds(g, 1), :] for g in range(_G)], axis=1)
    hpre = dinv * ht + b_ref[...].reshape(_D, 1)
    for g in range(_G):
        sl = slice(g * 128, (g + 1) * 128)
        h3_ref[:, g, :, :] = ht[:, sl].reshape(_D // 8, 8, 128)
        hp3_ref[:, g, :, :] = hpre[:, sl].reshape(_D // 8, 8, 128)


_PIB = lax.GatherScatterMode.PROMISE_IN_BOUNDS
_GDN = lax.GatherDimensionNumbers(
    offset_dims=(), collapsed_slice_dims=(0,), start_index_map=(0,))


def _splat(vec, j):
    idx = jnp.full((16, 1), j, jnp.int32)
    return lax.gather(vec, idx, _GDN, slice_sizes=(1,), mode=_PIB)


_FPW = 8


def _sc_agg_body(ht_hbm, hpret_hbm, src_hbm, dst_hbm, norm_hbm,
                 yt_hbm, src_v, dst_v, norm_v, hloc_v, acc_v, sem):
    c = lax.axis_index("c")
    s = lax.axis_index("s")
    w = s * 2 + c
    cps = [
        pltpu.async_copy(ht_hbm.at[w], hloc_v, sem),
        pltpu.async_copy(hpret_hbm.at[w], acc_v, sem),
        pltpu.async_copy(src_hbm, src_v, sem),
        pltpu.async_copy(dst_hbm, dst_v, sem),
        pltpu.async_copy(norm_hbm, norm_v, sem),
    ]
    for cp in cps:
        cp.wait()

    lane = lax.iota(jnp.int32, 16)
    lmask = lane < _FPW
    lrow = jnp.minimum(lane, _FPW - 1)
    for ev in range(_EP // 16):
        sl = pl.ds(ev * 16, 16)
        srcv = src_v[sl]
        dstv = dst_v[sl]
        normv = norm_v[sl]
        for j in range(16):
            sj = _splat(srcv, j)
            dj = _splat(dstv, j)
            nj = _splat(normv, j)
            g = plsc.load_gather(
                hloc_v, [sj >> 7, lrow, sj & 127], mask=lmask)
            plsc.addupdate_scatter(
                acc_v, [dj >> 7, lrow, dj & 127], g * nj, mask=lmask)

    pltpu.sync_copy(acc_v, yt_hbm.at[w])


_sc_agg = pl.kernel(
    _sc_agg_body,
    out_type=jax.ShapeDtypeStruct((_D // 8, _G, 8, 128), jnp.float32),
    mesh=plsc.VectorSubcoreMesh(core_axis_name="c", subcore_axis_name="s"),
    compiler_params=pltpu.CompilerParams(use_tc_tiling_on_sc=False,
                                         needs_layout_passes=False),
    scratch_types=[
        pltpu.VMEM((_EP,), jnp.int32),
        pltpu.VMEM((_EP,), jnp.int32),
        pltpu.VMEM((_EP,), jnp.float32),
        pltpu.VMEM((_G, _FPW, 128), jnp.float32),
        pltpu.VMEM((_G, _FPW, 128), jnp.float32),
        pltpu.SemaphoreType.DMA,
    ],
)

_tc_first = pl.pallas_call(
    _tc_first_body,
    out_shape=(
        jax.ShapeDtypeStruct((_D // 8, _G, 8, 128), jnp.float32),
        jax.ShapeDtypeStruct((_D // 8, _G, 8, 128), jnp.float32),
        jax.ShapeDtypeStruct((_G, 128), jnp.float32),
        jax.ShapeDtypeStruct((_EP,), jnp.float32),
        jax.ShapeDtypeStruct((_EP,), jnp.int32),
        jax.ShapeDtypeStruct((_EP,), jnp.int32),
    ),
)

_tc_mid = pl.pallas_call(
    _tc_mid_body,
    out_shape=(
        jax.ShapeDtypeStruct((_D // 8, _G, 8, 128), jnp.float32),
        jax.ShapeDtypeStruct((_D // 8, _G, 8, 128), jnp.float32),
    ),
)


def kernel(L_x_, L_edge_index_,
           L_self_modules_convs_modules_0_modules_lin_parameters_weight_,
           L_self_modules_convs_modules_0_parameters_bias_,
           L_self_modules_convs_modules_1_modules_lin_parameters_weight_,
           L_self_modules_convs_modules_1_parameters_bias_,
           L_self_modules_convs_modules_2_modules_lin_parameters_weight_,
           L_self_modules_convs_modules_2_parameters_bias_,
           L_self_modules_convs_modules_3_modules_lin_parameters_weight_,
           L_self_modules_convs_modules_3_parameters_bias_,
           L_self_modules_convs_modules_4_modules_lin_parameters_weight_,
           L_self_modules_convs_modules_4_parameters_bias_):
    ws = [L_self_modules_convs_modules_0_modules_lin_parameters_weight_,
          L_self_modules_convs_modules_1_modules_lin_parameters_weight_,
          L_self_modules_convs_modules_2_modules_lin_parameters_weight_,
          L_self_modules_convs_modules_3_modules_lin_parameters_weight_,
          L_self_modules_convs_modules_4_modules_lin_parameters_weight_]
    bs = [L_self_modules_convs_modules_0_parameters_bias_,
          L_self_modules_convs_modules_1_parameters_bias_,
          L_self_modules_convs_modules_2_parameters_bias_,
          L_self_modules_convs_modules_3_parameters_bias_,
          L_self_modules_convs_modules_4_parameters_bias_]

    e = L_edge_index_.astype(jnp.int32)
    eint = jnp.pad(e.T, ((0, 0), (0, 6)))

    h3, hp3, dinv8, norm, src, dst = _tc_first(eint, L_x_, ws[0], bs[0])
    yt = _sc_agg(h3, hp3, src, dst, norm)
    for i in range(1, 5):
        h3, hp3 = _tc_mid(yt, ws[i], bs[i], dinv8)
        yt = _sc_agg(h3, hp3, src, dst, norm)
    ytt = yt.transpose(0, 2, 1, 3).reshape(_D, _NP)
    return ytt[:, :_N].T
